# baseline (speedup 1.0000x reference)
import jax
import jax.numpy as jnp
from jax.experimental import pallas as pl
from jax.experimental.pallas import tpu as pltpu

B = 8
N = 20000
C = 80
TOPN = 1000
MIN_SCORE = 0.05
NMS_TH = 0.5
MAX_OBJ = 100


def _scores_body(cls_ref, s_ref, c_ref):
    x = cls_ref[0]
    smax = jnp.max(x, axis=-1)
    arg = jnp.argmax(x, axis=-1)
    s_ref[0, 0] = smax
    c_ref[0, 0] = arg.astype(jnp.int32)


def _scores_call(cls2):
    s, c = pl.pallas_call(
        _scores_body,
        grid=(B,),
        in_specs=[pl.BlockSpec((1, N, C), lambda r: (r, 0, 0))],
        out_specs=[
            pl.BlockSpec((1, 1, N), lambda r: (r, 0, 0)),
            pl.BlockSpec((1, 1, N), lambda r: (r, 0, 0)),
        ],
        out_shape=[
            jax.ShapeDtypeStruct((B, 1, N), jnp.float32),
            jax.ShapeDtypeStruct((B, 1, N), jnp.int32),
        ],
    )(cls2)
    return s.reshape(B, N), c.reshape(B, N)


def _decode_body(reg_ref, anc_ref, box_ref):
    a = anc_ref[0]
    r = reg_ref[0]
    wh_x = a[2:3] - a[0:1]
    wh_y = a[3:4] - a[1:2]
    ctr_x = a[0:1] + 0.5 * wh_x
    ctr_y = a[1:2] + 0.5 * wh_y
    pw_x = jnp.exp(r[2:3]) * wh_x
    pw_y = jnp.exp(r[3:4]) * wh_y
    pc_x = r[0:1] * wh_x + ctr_x
    pc_y = r[1:2] * wh_y + ctr_y
    mn_x = pc_x - 0.5 * pw_x
    mn_y = pc_y - 0.5 * pw_y
    mx_x = pc_x + 0.5 * pw_x
    mx_y = pc_y + 0.5 * pw_y
    box = jnp.concatenate([mn_x, mn_y, mx_x, mx_y], axis=0)
    box_ref[0] = box.astype(jnp.int32).astype(jnp.float32)


def _decode_call(regT, ancT):
    return pl.pallas_call(
        _decode_body,
        grid=(B,),
        in_specs=[
            pl.BlockSpec((1, 4, N), lambda r: (r, 0, 0)),
            pl.BlockSpec((1, 4, N), lambda r: (r, 0, 0)),
        ],
        out_specs=pl.BlockSpec((1, 4, N), lambda r: (r, 0, 0)),
        out_shape=jax.ShapeDtypeStruct((B, 4, N), jnp.float32),
    )(regT, ancT)


def _decode_one(scores, classes, boxes):
    m = scores > MIN_SCORE
    sort_key = jnp.where(m, -scores, jnp.inf)
    order = jnp.argsort(sort_key, stable=True)[:TOPN]
    s = scores[order]
    c = classes[order]
    b = boxes[order]
    v = m[order]
    wh = b[:, 2:4] - b[:, 0:2]
    areas = jnp.clip(wh[:, 0] * wh[:, 1], 0.0001, None)
    idxs = jnp.arange(TOPN)

    def body(i, suppressed):
        active = ~suppressed[i]
        tl = jnp.maximum(b[i, 0:2], b[:, 0:2])
        br = jnp.minimum(b[i, 2:4], b[:, 2:4])
        sz = jnp.clip(br - tl, 0, None)
        overlap = sz[:, 0] * sz[:, 1]
        union = jnp.clip(areas[i] + areas - overlap, 0.0001, None)
        ious = overlap / union
        new_supp = active & (ious >= NMS_TH) & (idxs > i)
        return suppressed | new_supp

    suppressed = jax.lax.fori_loop(0, TOPN, body, ~v)
    keepmask = ~suppressed
    num_keep = jnp.sum(keepmask)
    take = jnp.argsort((~keepmask).astype(jnp.int32), stable=True)[:MAX_OBJ]
    ok = jnp.arange(MAX_OBJ) < num_keep
    out_s = jnp.where(ok, s[take], jnp.float32(-1.0))
    out_c = jnp.where(ok, c[take], jnp.float32(-1.0))
    out_b = jnp.where(ok[:, None], b[take], jnp.float32(0.0))
    return out_s, out_c, out_b


def kernel(cls_heads, reg_heads, batch_anchors):
    cls2 = cls_heads.reshape(B, N, C)
    regT = reg_heads.reshape(B, N, 4).transpose(0, 2, 1)
    ancT = batch_anchors.reshape(B, N, 4).transpose(0, 2, 1)

    scores, classes = _scores_call(cls2)
    boxesT = _decode_call(regT, ancT)
    boxes = boxesT.transpose(0, 2, 1)

    return jax.vmap(_decode_one)(
        scores, classes.astype(jnp.float32), boxes)


# speedup vs baseline: 5.6198x; 5.6198x over previous
import functools

import jax
import jax.numpy as jnp
from jax import lax
from jax.experimental import pallas as pl
from jax.experimental.pallas import tpu as pltpu
from jax.experimental.pallas import tpu_sc as plsc

B = 8
N = 20000
C = 80
TOPN = 1000
MIN_SCORE = 0.05
NMS_TH = 0.5
MAX_OBJ = 100
NP_ = 1024
NW = NP_ // 16


def _scores_body(cls_ref, s_ref, c_ref):
    x = cls_ref[0]
    smax = jnp.max(x, axis=-1)
    arg = jnp.argmax(x, axis=-1)
    s_ref[0, 0] = smax
    c_ref[0, 0] = arg.astype(jnp.int32)


def _scores_call(cls2):
    s, c = pl.pallas_call(
        _scores_body,
        grid=(B,),
        in_specs=[pl.BlockSpec((1, N, C), lambda r: (r, 0, 0))],
        out_specs=[
            pl.BlockSpec((1, 1, N), lambda r: (r, 0, 0)),
            pl.BlockSpec((1, 1, N), lambda r: (r, 0, 0)),
        ],
        out_shape=[
            jax.ShapeDtypeStruct((B, 1, N), jnp.float32),
            jax.ShapeDtypeStruct((B, 1, N), jnp.int32),
        ],
    )(cls2)
    return s.reshape(B, N), c.reshape(B, N)


def _decode_body(reg_ref, anc_ref, box_ref):
    a = anc_ref[0]
    r = reg_ref[0]
    wh_x = a[2:3] - a[0:1]
    wh_y = a[3:4] - a[1:2]
    ctr_x = a[0:1] + 0.5 * wh_x
    ctr_y = a[1:2] + 0.5 * wh_y
    pw_x = jnp.exp(r[2:3]) * wh_x
    pw_y = jnp.exp(r[3:4]) * wh_y
    pc_x = r[0:1] * wh_x + ctr_x
    pc_y = r[1:2] * wh_y + ctr_y
    mn_x = pc_x - 0.5 * pw_x
    mn_y = pc_y - 0.5 * pw_y
    mx_x = pc_x + 0.5 * pw_x
    mx_y = pc_y + 0.5 * pw_y
    box = jnp.concatenate([mn_x, mn_y, mx_x, mx_y], axis=0)
    box_ref[0] = box.astype(jnp.int32).astype(jnp.float32)


def _decode_call(regT, ancT):
    return pl.pallas_call(
        _decode_body,
        grid=(B,),
        in_specs=[
            pl.BlockSpec((1, 4, N), lambda r: (r, 0, 0)),
            pl.BlockSpec((1, 4, N), lambda r: (r, 0, 0)),
        ],
        out_specs=pl.BlockSpec((1, 4, N), lambda r: (r, 0, 0)),
        out_shape=jax.ShapeDtypeStruct((B, 4, N), jnp.float32),
    )(regT, ancT)


def _pack_matrix():
    jj = lax.broadcasted_iota(jnp.int32, (NP_, NW), 0)
    ww = lax.broadcasted_iota(jnp.int32, (NP_, NW), 1)
    val = jnp.where(jj // 16 == ww, (1 << (jj % 16)), 0)
    return val.astype(jnp.bfloat16)


def _iou_body(s_ref, b_ref, bT_ref, mp_ref, supp0_ref):
    P = _pack_matrix()
    x1r = bT_ref[:, 0:1, :]
    y1r = bT_ref[:, 1:2, :]
    x2r = bT_ref[:, 2:3, :]
    y2r = bT_ref[:, 3:4, :]
    arear = jnp.clip((x2r - x1r) * (y2r - y1r), 0.0001, None)

    inv = s_ref[:] <= MIN_SCORE
    supp0_ref[:] = jnp.dot(inv.astype(jnp.bfloat16), P,
                           preferred_element_type=jnp.float32).astype(jnp.int32)

    BK = 128
    for k in range(NP_ // BK):
        sl = pl.ds(k * BK, BK)
        x1c = b_ref[:, sl, 0:1]
        y1c = b_ref[:, sl, 1:2]
        x2c = b_ref[:, sl, 2:3]
        y2c = b_ref[:, sl, 3:4]
        areac = jnp.clip((x2c - x1c) * (y2c - y1c), 0.0001, None)
        szx = jnp.clip(jnp.minimum(x2c, x2r) - jnp.maximum(x1c, x1r), 0, None)
        szy = jnp.clip(jnp.minimum(y2c, y2r) - jnp.maximum(y1c, y1r), 0, None)
        ov = szx * szy
        un = jnp.clip(areac + arear - ov, 0.0001, None)
        iou = ov / un
        jglob = lax.broadcasted_iota(jnp.int32, (B, BK, NP_), 2)
        iglob = lax.broadcasted_iota(jnp.int32, (B, BK, NP_), 1) + k * BK
        Mb = (iou >= NMS_TH) & (jglob > iglob)
        W = jnp.dot(Mb.reshape(B * BK, NP_).astype(jnp.bfloat16), P,
                    preferred_element_type=jnp.float32)
        mp_ref[:, sl, :] = W.reshape(B, BK, NW).astype(jnp.int32)


def _iou_call(s_sorted, b_sorted, bT_sorted):
    return pl.pallas_call(
        _iou_body,
        in_specs=[
            pl.BlockSpec((B, NP_), lambda: (0, 0)),
            pl.BlockSpec((B, NP_, 4), lambda: (0, 0, 0)),
            pl.BlockSpec((B, 4, NP_), lambda: (0, 0, 0)),
        ],
        out_specs=[
            pl.BlockSpec((B, NP_, NW), lambda: (0, 0, 0)),
            pl.BlockSpec((B, NW), lambda: (0, 0)),
        ],
        out_shape=[
            jax.ShapeDtypeStruct((B, NP_, NW), jnp.int32),
            jax.ShapeDtypeStruct((B, NW), jnp.int32),
        ],
    )(s_sorted, b_sorted, bT_sorted)


def _nms_seq_kernel():
    info = plsc.get_sparse_core_info()
    nc = info.num_cores

    mesh = plsc.VectorSubcoreMesh(core_axis_name="c", subcore_axis_name="s")

    @functools.partial(
        pl.kernel,
        mesh=mesh,
        compiler_params=pltpu.CompilerParams(needs_layout_passes=False),
        out_type=[
            jax.ShapeDtypeStruct((B, 128), jnp.float32),
            jax.ShapeDtypeStruct((B, 128), jnp.float32),
            jax.ShapeDtypeStruct((B, 512), jnp.float32),
        ],
        scratch_types=[
            pltpu.VMEM((NP_ * NW,), jnp.int32),
            pltpu.VMEM((NP_ + 16,), jnp.float32),
            pltpu.VMEM((NP_ + 16,), jnp.float32),
            pltpu.VMEM((4 * NP_ + 16,), jnp.float32),
            pltpu.VMEM((128,), jnp.int32),
            pltpu.VMEM((128,), jnp.float32),
            pltpu.VMEM((128,), jnp.float32),
            pltpu.VMEM((512,), jnp.float32),
        ],
    )
    def k(mp_hbm, supp0_hbm, s_hbm, c_hbm, b_hbm,
          so_hbm, co_hbm, bo_hbm,
          Mv, sv, cv, bv, suppv, sov, cov, bov):
        wid = lax.axis_index("s") * nc + lax.axis_index("c")
        lane = lax.iota(jnp.int32, 16)

        @pl.when(wid < B)
        def _():
            r = wid
            pltpu.sync_copy(mp_hbm.at[r], Mv)
            pltpu.sync_copy(s_hbm.at[r], sv.at[pl.ds(0, NP_)])
            pltpu.sync_copy(c_hbm.at[r], cv.at[pl.ds(0, NP_)])
            pltpu.sync_copy(b_hbm.at[r], bv.at[pl.ds(0, 4 * NP_)])
            pltpu.sync_copy(supp0_hbm.at[pl.ds(r * NW, NW)],
                            suppv.at[pl.ds(0, NW)])

            def _bitvec(i):
                wvec = plsc.load_gather(
                    suppv, [jnp.full((16,), i // 16, jnp.int32)])
                return lax.shift_right_logical(wvec, i % 16) & 1

            def body(i, carry):
                msk = _bitvec(i) - 1
                for v in range(NW // 16):
                    sl = pl.ds(v * 16, 16)
                    suppv[sl] = suppv[sl] | (Mv[pl.ds(i * NW + v * 16, 16)] & msk)
                return carry

            lax.fori_loop(0, NP_, body, 0)

            for v in range(8):
                sov[pl.ds(v * 16, 16)] = jnp.full((16,), -1.0, jnp.float32)
                cov[pl.ds(v * 16, 16)] = jnp.full((16,), -1.0, jnp.float32)
            for v in range(32):
                bov[pl.ds(v * 16, 16)] = jnp.zeros((16,), jnp.float32)

            def body2(i, cnt):
                bit0 = _bitvec(i)[0]
                pred = (bit0 == 0) & (cnt < MAX_OBJ)
                cntv = jnp.full((16,), cnt, jnp.int32)
                plsc.store_scatter(sov, [cntv], sv[pl.ds(i, 16)],
                                   mask=(lane == 0) & pred)
                plsc.store_scatter(cov, [cntv], cv[pl.ds(i, 16)],
                                   mask=(lane == 0) & pred)
                plsc.store_scatter(bov, [4 * cntv + lane],
                                   bv[pl.ds(4 * i, 16)],
                                   mask=(lane < 4) & pred)
                return cnt + (1 - bit0)

            lax.fori_loop(0, NP_, body2, 0)

            pltpu.sync_copy(sov, so_hbm.at[r])
            pltpu.sync_copy(cov, co_hbm.at[r])
            pltpu.sync_copy(bov, bo_hbm.at[r])

    return k


def _decode_one(scores, classes, boxes):
    m = scores > MIN_SCORE
    sort_key = jnp.where(m, -scores, jnp.inf)
    order = jnp.argsort(sort_key, stable=True)[:TOPN]
    s = scores[order]
    c = classes[order]
    b = boxes[order]
    v = m[order]
    wh = b[:, 2:4] - b[:, 0:2]
    areas = jnp.clip(wh[:, 0] * wh[:, 1], 0.0001, None)
    idxs = jnp.arange(TOPN)

    def body(i, suppressed):
        active = ~suppressed[i]
        tl = jnp.maximum(b[i, 0:2], b[:, 0:2])
        br = jnp.minimum(b[i, 2:4], b[:, 2:4])
        sz = jnp.clip(br - tl, 0, None)
        overlap = sz[:, 0] * sz[:, 1]
        union = jnp.clip(areas[i] + areas - overlap, 0.0001, None)
        ious = overlap / union
        new_supp = active & (ious >= NMS_TH) & (idxs > i)
        return suppressed | new_supp

    suppressed = jax.lax.fori_loop(0, TOPN, body, ~v)
    keepmask = ~suppressed
    num_keep = jnp.sum(keepmask)
    take = jnp.argsort((~keepmask).astype(jnp.int32), stable=True)[:MAX_OBJ]
    ok = jnp.arange(MAX_OBJ) < num_keep
    out_s = jnp.where(ok, s[take], jnp.float32(-1.0))
    out_c = jnp.where(ok, c[take], jnp.float32(-1.0))
    out_b = jnp.where(ok[:, None], b[take], jnp.float32(0.0))
    return out_s, out_c, out_b


def kernel(cls_heads, reg_heads, batch_anchors):
    cls2 = cls_heads.reshape(B, N, C)
    regT = reg_heads.reshape(B, N, 4).transpose(0, 2, 1)
    ancT = batch_anchors.reshape(B, N, 4).transpose(0, 2, 1)

    scores, classes = _scores_call(cls2)
    boxesT = _decode_call(regT, ancT)
    boxes = boxesT.transpose(0, 2, 1)

    m = scores > MIN_SCORE
    sort_key = jnp.where(m, -scores, jnp.inf)
    order = jnp.argsort(sort_key, axis=1, stable=True)[:, :TOPN]
    s_sorted = jnp.take_along_axis(scores, order, axis=1)
    c_sorted = jnp.take_along_axis(classes, order, axis=1).astype(jnp.float32)
    b_sorted = jnp.take_along_axis(boxes, order[:, :, None], axis=1)

    pad = NP_ - TOPN
    s_sorted = jnp.pad(s_sorted, ((0, 0), (0, pad)), constant_values=-1.0)
    c_sorted = jnp.pad(c_sorted, ((0, 0), (0, pad)))
    b_sorted = jnp.pad(b_sorted, ((0, 0), (0, pad), (0, 0)))
    bT_sorted = b_sorted.transpose(0, 2, 1)

    mp, supp0 = _iou_call(s_sorted, b_sorted, bT_sorted)
    so, co, bo = _nms_seq_kernel()(mp.reshape(B, NP_ * NW),
                                   supp0.reshape(B * NW),
                                   s_sorted, c_sorted,
                                   b_sorted.reshape(B, 4 * NP_))
    return (so[:, :MAX_OBJ], co[:, :MAX_OBJ],
            bo.reshape(B, 128, 4)[:, :MAX_OBJ])
